# Initial kernel scaffold; baseline (speedup 1.0000x reference)
import jax
import jax.numpy as jnp
from jax.experimental import pallas as pl


def kernel(x, W1, b1, W2, b2, emb, W3, b3, W4, b4):
    raise NotImplementedError("write your pallas kernel here")


# speedup vs baseline: 7.6419x; 7.6419x over previous
import functools

import jax
import jax.numpy as jnp
from jax import lax
from jax.experimental import pallas as pl
from jax.experimental.pallas import tpu as pltpu
from jax.experimental.pallas import tpu_sc as plsc

BATCH = 1024
STATE_DIM = 768
LATENT_DIM = 256
NUM_EMB = 1024
HIDDEN = 64

_HI = lax.Precision.HIGHEST


def _enc_body(x_ref, w1_ref, b1_ref, w2_ref, b2_ref, emb_ref, z_e_ref, idx_ref):
    x = x_ref[...]
    h = jnp.maximum(
        lax.dot_general(x, w1_ref[...], (((1,), (0,)), ((), ())))
        + b1_ref[...],
        0.0,
    )
    z_e = (
        lax.dot_general(h, w2_ref[...], (((1,), (0,)), ((), ())))
        + b2_ref[...]
    )
    z_e_ref[...] = z_e
    emb = emb_ref[...]
    cross = lax.dot_general(z_e, emb, (((1,), (1,)), ((), ())), precision=_HI)
    ones = jnp.ones((1, LATENT_DIM), jnp.float32)
    norms = lax.dot_general(
        ones, emb * emb, (((1,), (1,)), ((), ())), precision=_HI
    )
    scores = norms - 2.0 * cross
    m = jnp.min(scores, axis=1, keepdims=True)
    iota = lax.broadcasted_iota(jnp.int32, (BATCH, NUM_EMB), 1)
    idx = jnp.min(
        jnp.where(scores <= m, iota, NUM_EMB), axis=1, keepdims=True
    )
    idx_ref[...] = idx


def _dec_body(z_q_ref, w3_ref, b3_ref, w4_ref, b4_ref, out_ref):
    h2 = jnp.maximum(
        lax.dot_general(z_q_ref[...], w3_ref[...], (((1,), (0,)), ((), ())))
        + b3_ref[...],
        0.0,
    )
    out_ref[...] = (
        lax.dot_general(h2, w4_ref[...], (((1,), (0,)), ((), ())))
        + b4_ref[...]
    )


_NC = 2
_NS = 16
_NW = _NC * _NS
_BPW = BATCH // _NW


def _make_sc_gather():
    mesh = plsc.VectorSubcoreMesh(core_axis_name="c", subcore_axis_name="s")

    @functools.partial(
        pl.kernel,
        mesh=mesh,
        out_type=jax.ShapeDtypeStruct((BATCH, LATENT_DIM), jnp.float32),
        scratch_types=[
            pltpu.VMEM((_BPW,), jnp.int32),
            pltpu.VMEM((_BPW, LATENT_DIM), jnp.float32),
            pltpu.SemaphoreType.DMA,
        ],
    )
    def _sc_gather(emb_hbm, idx_hbm, out_hbm, idx_v, rows_v, sem):
        wid = lax.axis_index("s") * _NC + lax.axis_index("c")
        base = wid * _BPW
        pltpu.sync_copy(idx_hbm.at[pl.ds(base, _BPW)], idx_v)
        pltpu.async_copy(emb_hbm.at[idx_v], rows_v, sem).wait()
        pltpu.sync_copy(rows_v, out_hbm.at[pl.ds(base, _BPW)])

    return _sc_gather


def kernel(x, W1, b1, W2, b2, emb, W3, b3, W4, b4):
    z_e, idx2 = pl.pallas_call(
        _enc_body,
        out_shape=[
            jax.ShapeDtypeStruct((BATCH, LATENT_DIM), jnp.float32),
            jax.ShapeDtypeStruct((BATCH, 1), jnp.int32),
        ],
    )(x, W1, b1.reshape(1, HIDDEN), W2, b2.reshape(1, LATENT_DIM), emb)
    z_q = _make_sc_gather()(emb, idx2.reshape(BATCH))
    x_recon = pl.pallas_call(
        _dec_body,
        out_shape=jax.ShapeDtypeStruct((BATCH, STATE_DIM), jnp.float32),
    )(z_q, W3, b3.reshape(1, HIDDEN), W4, b4.reshape(1, STATE_DIM))
    return (x_recon, z_e, z_q)
